# Initial kernel scaffold; baseline (speedup 1.0000x reference)
import jax
import jax.numpy as jnp
from jax.experimental import pallas as pl


def kernel(input, pos):
    raise NotImplementedError("write your pallas kernel here")


# speedup vs baseline: 1.6761x; 1.6761x over previous
import functools

import jax
import jax.numpy as jnp
from jax import lax
from jax.experimental import pallas as pl
from jax.experimental.pallas import tpu as pltpu
from jax.experimental.pallas import tpu_sc as plsc

WINDOW = 127
XPAD = 136
LANES = 16


def _sc_body(C, pairs_per_worker, num_cores,
             in_hbm, pos_hbm, out_hbm, pos_v, slab, stage):
    wid = lax.axis_index("s") * num_cores + lax.axis_index("c")
    pltpu.sync_copy(pos_hbm, pos_v)
    W = in_hbm.shape[3]
    iota = lax.iota(jnp.int32, LANES)

    def scalar_at(k):
        splat = plsc.load_gather(pos_v, [jnp.full((LANES,), k, jnp.int32)])
        return jnp.max(splat)

    def step(t, carry):
        pair = wid * pairs_per_worker + t
        b = pair // C
        c = pair % C
        y = scalar_at(2 * b)
        x = scalar_at(2 * b + 1)
        xb = pl.multiple_of(lax.min((x // 8) * 8, jnp.int32(W - XPAD)), 8)
        dx = x - xb
        pltpu.sync_copy(
            in_hbm.at[b, c, pl.ds(y, WINDOW), pl.ds(xb, XPAD)], slab)
        col0 = iota + dx

        def row(i, cc):
            row_idx = jnp.full((LANES,), i, jnp.int32)
            for j in range(7):
                v = plsc.load_gather(slab, [row_idx, col0 + (j * LANES)])
                stage[i, pl.ds(j * LANES, LANES)] = v
            v = plsc.load_gather(slab, [row_idx, col0 + (7 * LANES)])
            plsc.store_scatter(
                stage, [row_idx, iota + (7 * LANES)], v, mask=iota < 15)
            return cc

        lax.fori_loop(0, WINDOW, row, 0)
        pltpu.sync_copy(stage, out_hbm.at[b, c])
        return carry

    lax.fori_loop(0, pairs_per_worker, step, 0)


def kernel(input, pos):
    B, C, H, W = input.shape
    info = plsc.get_sparse_core_info()
    num_workers = info.num_cores * info.num_subcores
    pairs_per_worker = (B * C) // num_workers
    mesh = plsc.VectorSubcoreMesh(core_axis_name="c", subcore_axis_name="s")
    run = pl.kernel(
        functools.partial(_sc_body, C, pairs_per_worker, info.num_cores),
        out_type=jax.ShapeDtypeStruct((B, C, WINDOW, WINDOW), input.dtype),
        mesh=mesh,
        scratch_types=[
            pltpu.VMEM((2 * B,), jnp.int32),
            pltpu.VMEM((WINDOW, XPAD), jnp.float32),
            pltpu.VMEM((WINDOW, WINDOW), jnp.float32),
        ],
        compiler_params=pltpu.CompilerParams(
            use_tc_tiling_on_sc=False, needs_layout_passes=False),
    )
    return run(input, pos.astype(jnp.int32).reshape(-1))


# speedup vs baseline: 1.9455x; 1.1607x over previous
import functools

import jax
import jax.numpy as jnp
from jax import lax
from jax.experimental import pallas as pl
from jax.experimental.pallas import tpu as pltpu
from jax.experimental.pallas import tpu_sc as plsc

WINDOW = 127
XPAD = 136
LANES = 16
NCHUNK = 8


def _sc_body(C, pairs_per_worker, num_cores,
             in_hbm, pos_hbm, out_hbm, pos_v, slab, stage, in_sem, out_sem):
    wid = lax.axis_index("s") * num_cores + lax.axis_index("c")
    pltpu.sync_copy(pos_hbm, pos_v)
    W = in_hbm.shape[3]
    iota = lax.iota(jnp.int32, LANES)

    def scalar_at(k):
        splat = plsc.load_gather(pos_v, [jnp.full((LANES,), k, jnp.int32)])
        return jnp.max(splat)

    def coords(t):
        pair = wid * pairs_per_worker + t
        return pair // C, pair % C

    def window(t):
        b, c = coords(t)
        y = scalar_at(2 * b)
        x = scalar_at(2 * b + 1)
        xb = pl.multiple_of(lax.min((x // 8) * 8, jnp.int32(W - XPAD)), 8)
        return b, c, y, xb, x - xb

    def start_in(t, k):
        b, c, y, xb, _ = window(t)
        pltpu.make_async_copy(
            in_hbm.at[b, c, pl.ds(y, WINDOW), pl.ds(xb, XPAD)],
            slab.at[k], in_sem.at[k]).start()

    def wait_in(k):
        pltpu.make_async_copy(
            in_hbm.at[0, 0, pl.ds(0, WINDOW), pl.ds(0, XPAD)],
            slab.at[k], in_sem.at[k]).wait()

    def start_out(t, k):
        b, c = coords(t)
        pltpu.make_async_copy(
            stage.at[k], out_hbm.at[b, c], out_sem.at[k]).start()

    def wait_out(k):
        pltpu.make_async_copy(
            stage.at[k], out_hbm.at[0, 0], out_sem.at[k]).wait()

    def shift(t, k):
        _, _, _, _, dx = window(t)
        col0 = iota + dx
        ksplat = jnp.full((LANES,), k, jnp.int32)

        @plsc.parallel_loop(0, WINDOW, unroll=4)
        def _row(i):
            row_idx = jnp.full((LANES,), i, jnp.int32)
            vals = [
                plsc.load_gather(slab, [ksplat, row_idx, col0 + j * LANES])
                for j in range(NCHUNK)
            ]
            for j in range(NCHUNK - 1):
                stage[k, i, pl.ds(j * LANES, LANES)] = vals[j]
            plsc.store_scatter(
                stage, [ksplat, row_idx, iota + (NCHUNK - 1) * LANES],
                vals[NCHUNK - 1], mask=iota < LANES - 1)

    start_in(0, 0)
    start_in(1, 1)

    def step2(u, carry):
        for parity in range(2):
            t = 2 * u + parity
            wait_in(parity)

            @pl.when(t >= 2)
            def _():
                wait_out(parity)

            shift(t, parity)
            start_out(t, parity)

            @pl.when(t + 2 < pairs_per_worker)
            def _():
                start_in(t + 2, parity)

        return carry

    lax.fori_loop(0, pairs_per_worker // 2, step2, 0)
    wait_out(0)
    wait_out(1)


def kernel(input, pos):
    B, C, H, W = input.shape
    info = plsc.get_sparse_core_info()
    num_workers = info.num_cores * info.num_subcores
    pairs_per_worker = (B * C) // num_workers
    mesh = plsc.VectorSubcoreMesh(core_axis_name="c", subcore_axis_name="s")
    run = pl.kernel(
        functools.partial(_sc_body, C, pairs_per_worker, info.num_cores),
        out_type=jax.ShapeDtypeStruct((B, C, WINDOW, WINDOW), input.dtype),
        mesh=mesh,
        scratch_types=[
            pltpu.VMEM((2 * B,), jnp.int32),
            pltpu.VMEM((2, WINDOW, XPAD), jnp.float32),
            pltpu.VMEM((2, WINDOW, WINDOW), jnp.float32),
            pltpu.SemaphoreType.DMA((2,)),
            pltpu.SemaphoreType.DMA((2,)),
        ],
        compiler_params=pltpu.CompilerParams(
            use_tc_tiling_on_sc=False, needs_layout_passes=False),
    )
    return run(input, pos.astype(jnp.int32).reshape(-1))


# speedup vs baseline: 1.9506x; 1.0026x over previous
import functools

import jax
import jax.numpy as jnp
from jax import lax
from jax.experimental import pallas as pl
from jax.experimental.pallas import tpu as pltpu
from jax.experimental.pallas import tpu_sc as plsc

WINDOW = 127
XPAD = 136
LANES = 16
NCHUNK = 8


def _sc_body(C, pairs_per_worker, num_cores,
             in_hbm, pos_hbm, out_hbm, pos_v, slab, stage, in_sem, out_sem):
    wid = lax.axis_index("s") * num_cores + lax.axis_index("c")
    pltpu.sync_copy(pos_hbm, pos_v)
    W = in_hbm.shape[3]
    iota = lax.iota(jnp.int32, LANES)

    def scalar_at(k):
        splat = plsc.load_gather(pos_v, [jnp.full((LANES,), k, jnp.int32)])
        return jnp.max(splat)

    def coords(t):
        pair = wid * pairs_per_worker + t
        return pair // C, pair % C

    def window(t):
        b, c = coords(t)
        y = scalar_at(2 * b)
        x = scalar_at(2 * b + 1)
        xb = pl.multiple_of(lax.min((x // 8) * 8, jnp.int32(W - XPAD)), 8)
        return b, c, y, xb, x - xb

    def start_in(t, k):
        b, c, y, xb, _ = window(t)
        pltpu.make_async_copy(
            in_hbm.at[b, c, pl.ds(y, WINDOW), pl.ds(xb, XPAD)],
            slab.at[k], in_sem.at[k]).start()

    def wait_in(k):
        pltpu.make_async_copy(
            in_hbm.at[0, 0, pl.ds(0, WINDOW), pl.ds(0, XPAD)],
            slab.at[k], in_sem.at[k]).wait()

    def start_out(t, k):
        b, c = coords(t)
        pltpu.make_async_copy(
            stage.at[k], out_hbm.at[b, c], out_sem.at[k]).start()

    def wait_out(k):
        pltpu.make_async_copy(
            stage.at[k], out_hbm.at[0, 0], out_sem.at[k]).wait()

    def shift(t, k):
        _, _, _, _, dx = window(t)

        @plsc.parallel_loop(0, WINDOW, unroll=4)
        def _row(i):
            offs = [j * LANES for j in range(NCHUNK - 1)] + [WINDOW - LANES]
            vals = [slab[k, i, pl.ds(dx + o, LANES)] for o in offs]
            for o, v in zip(offs, vals):
                stage[k, i, pl.ds(o, LANES)] = v

    start_in(0, 0)
    start_in(1, 1)

    def step2(u, carry):
        for parity in range(2):
            t = 2 * u + parity
            wait_in(parity)

            @pl.when(t >= 2)
            def _():
                wait_out(parity)

            shift(t, parity)
            start_out(t, parity)

            @pl.when(t + 2 < pairs_per_worker)
            def _():
                start_in(t + 2, parity)

        return carry

    lax.fori_loop(0, pairs_per_worker // 2, step2, 0)
    wait_out(0)
    wait_out(1)


def kernel(input, pos):
    B, C, H, W = input.shape
    info = plsc.get_sparse_core_info()
    num_workers = info.num_cores * info.num_subcores
    pairs_per_worker = (B * C) // num_workers
    mesh = plsc.VectorSubcoreMesh(core_axis_name="c", subcore_axis_name="s")
    run = pl.kernel(
        functools.partial(_sc_body, C, pairs_per_worker, info.num_cores),
        out_type=jax.ShapeDtypeStruct((B, C, WINDOW, WINDOW), input.dtype),
        mesh=mesh,
        scratch_types=[
            pltpu.VMEM((2 * B,), jnp.int32),
            pltpu.VMEM((2, WINDOW, XPAD), jnp.float32),
            pltpu.VMEM((2, WINDOW, WINDOW), jnp.float32),
            pltpu.SemaphoreType.DMA((2,)),
            pltpu.SemaphoreType.DMA((2,)),
        ],
        compiler_params=pltpu.CompilerParams(
            use_tc_tiling_on_sc=False, needs_layout_passes=False),
    )
    return run(input, pos.astype(jnp.int32).reshape(-1))
